# baseline (speedup 1.0000x reference)
import jax
import jax.numpy as jnp
from jax.experimental import pallas as pl
from jax.experimental.pallas import tpu as pltpu

_N = 10000
_BLK = 2000


_SPLIT = 8


def _fused_body(x_ref, w1, b1, w2, b2, w3, b3, wz, bz,
                wd1, bd1, wd2, bd2, wd3, bd3, wxb, bxb, c_ref,
                xbar_ref, q_ref, z_ref):
    f32 = jnp.float32
    H = _BLK // _SPLIT

    def mm(a, w):
        return jnp.dot(a, w[...], preferred_element_type=f32)

    def layer(hs, w, b):
        return [jnp.maximum(mm(h, w) + b[...], 0.0) for h in hs]

    c = c_ref[...]
    cc = jnp.sum(c * c, axis=1)[None, :]

    def soft_assign(z):
        zz = jnp.sum(z * z, axis=1, keepdims=True)
        cross = jnp.dot(z, c.T, preferred_element_type=f32)
        dist2 = zz - 2.0 * cross + cc
        q = 1.0 / (1.0 + dist2)
        return q / jnp.sum(q, axis=1, keepdims=True)

    sl = [slice(i * H, (i + 1) * H) for i in range(_SPLIT)]
    stages = [
        lambda h, _: jnp.maximum(mm(h, w1) + b1[...], 0.0),
        lambda h, _: jnp.maximum(mm(h, w2) + b2[...], 0.0),
        lambda h, _: jnp.maximum(mm(h, w3) + b3[...], 0.0),
        lambda h, _: mm(h, wz) + bz[...],
        lambda z, s: (z_ref.__setitem__((s, slice(None)), z),
                      q_ref.__setitem__((s, slice(None)), soft_assign(z)),
                      jnp.maximum(mm(z, wd1) + bd1[...], 0.0))[-1],
        lambda h, _: jnp.maximum(mm(h, wd2) + bd2[...], 0.0),
        lambda h, _: jnp.maximum(mm(h, wd3) + bd3[...], 0.0),
        lambda h, s: xbar_ref.__setitem__((s, slice(None)), mm(h, wxb) + bxb[...]),
    ]
    n_stages = len(stages)
    vals = [x_ref[s, :] for s in sl]
    for t in range(n_stages + _SPLIT - 1):
        for i in range(_SPLIT):
            stage = t - i
            if 0 <= stage < n_stages:
                vals[i] = stages[stage](vals[i], sl[i])


def _full(shape):
    return pl.BlockSpec(shape, lambda i: (0, 0))


def kernel(x, enc1_w, enc1_b, enc2_w, enc2_b, enc3_w, enc3_b, z_w, z_b,
           dec1_w, dec1_b, dec2_w, dec2_b, dec3_w, dec3_b, xbar_w, xbar_b,
           cluster):
    n, dx = x.shape
    nc, nz = cluster.shape
    biases = [enc1_b, enc2_b, enc3_b, z_b, dec1_b, dec2_b, dec3_b, xbar_b]
    b2d = [b.reshape(1, -1) for b in biases]
    weights = [enc1_w, enc2_w, enc3_w, z_w, dec1_w, dec2_w, dec3_w, xbar_w]

    in_specs = [pl.BlockSpec((_BLK, dx), lambda i: (i, 0))]
    for w, b in zip(weights, b2d):
        in_specs.append(_full(w.shape))
        in_specs.append(_full(b.shape))
    in_specs.append(_full(cluster.shape))

    out_specs = (
        pl.BlockSpec((_BLK, dx), lambda i: (i, 0)),
        pl.BlockSpec((_BLK, nc), lambda i: (i, 0)),
        pl.BlockSpec((_BLK, nz), lambda i: (i, 0)),
    )
    out_shape = (
        jax.ShapeDtypeStruct((n, dx), jnp.float32),
        jax.ShapeDtypeStruct((n, nc), jnp.float32),
        jax.ShapeDtypeStruct((n, nz), jnp.float32),
    )

    args = [x]
    for w, b in zip(weights, b2d):
        args.append(w)
        args.append(b)
    args.append(cluster)

    return pl.pallas_call(
        _fused_body,
        grid=(n // _BLK,),
        in_specs=in_specs,
        out_specs=out_specs,
        out_shape=out_shape,
        compiler_params=pltpu.CompilerParams(
            dimension_semantics=("parallel",),
            vmem_limit_bytes=110 * 1024 * 1024,
        ),
    )(*args)


# speedup vs baseline: 1.0341x; 1.0341x over previous
import jax
import jax.numpy as jnp
from jax.experimental import pallas as pl
from jax.experimental.pallas import tpu as pltpu

_N = 10000
_BLK = 2000


_SPLIT = 5


def _fused_body(x_ref, w1, b1, w2, b2, w3, b3, wz, bz,
                wd1, bd1, wd2, bd2, wd3, bd3, wxb, bxb, c_ref,
                xbar_ref, q_ref, z_ref):
    f32 = jnp.float32
    H = _BLK // _SPLIT

    def mm(a, w):
        return jnp.dot(a, w[...], preferred_element_type=f32)

    def layer(hs, w, b):
        return [jnp.maximum(mm(h, w) + b[...], 0.0) for h in hs]

    c = c_ref[...]
    cc = jnp.sum(c * c, axis=1)[None, :]

    def soft_assign(z):
        zz = jnp.sum(z * z, axis=1, keepdims=True)
        cross = jnp.dot(z, c.T, preferred_element_type=f32)
        dist2 = zz - 2.0 * cross + cc
        q = 1.0 / (1.0 + dist2)
        return q / jnp.sum(q, axis=1, keepdims=True)

    sl = [slice(i * H, (i + 1) * H) for i in range(_SPLIT)]
    stages = [
        lambda h, _: jnp.maximum(mm(h, w1) + b1[...], 0.0),
        lambda h, _: jnp.maximum(mm(h, w2) + b2[...], 0.0),
        lambda h, _: jnp.maximum(mm(h, w3) + b3[...], 0.0),
        lambda h, _: mm(h, wz) + bz[...],
        lambda z, s: (z_ref.__setitem__((s, slice(None)), z),
                      q_ref.__setitem__((s, slice(None)), soft_assign(z)),
                      jnp.maximum(mm(z, wd1) + bd1[...], 0.0))[-1],
        lambda h, _: jnp.maximum(mm(h, wd2) + bd2[...], 0.0),
        lambda h, _: jnp.maximum(mm(h, wd3) + bd3[...], 0.0),
        lambda h, s: xbar_ref.__setitem__((s, slice(None)), mm(h, wxb) + bxb[...]),
    ]
    n_stages = len(stages)
    vals = [x_ref[s, :] for s in sl]
    for t in range(n_stages + _SPLIT - 1):
        for i in range(_SPLIT):
            stage = t - i
            if 0 <= stage < n_stages:
                vals[i] = stages[stage](vals[i], sl[i])


def _full(shape):
    return pl.BlockSpec(shape, lambda i: (0, 0))


def kernel(x, enc1_w, enc1_b, enc2_w, enc2_b, enc3_w, enc3_b, z_w, z_b,
           dec1_w, dec1_b, dec2_w, dec2_b, dec3_w, dec3_b, xbar_w, xbar_b,
           cluster):
    n, dx = x.shape
    nc, nz = cluster.shape
    biases = [enc1_b, enc2_b, enc3_b, z_b, dec1_b, dec2_b, dec3_b, xbar_b]
    b2d = [b.reshape(1, -1) for b in biases]
    weights = [enc1_w, enc2_w, enc3_w, z_w, dec1_w, dec2_w, dec3_w, xbar_w]

    in_specs = [pl.BlockSpec((_BLK, dx), lambda i: (i, 0))]
    for w, b in zip(weights, b2d):
        in_specs.append(_full(w.shape))
        in_specs.append(_full(b.shape))
    in_specs.append(_full(cluster.shape))

    out_specs = (
        pl.BlockSpec((_BLK, dx), lambda i: (i, 0)),
        pl.BlockSpec((_BLK, nc), lambda i: (i, 0)),
        pl.BlockSpec((_BLK, nz), lambda i: (i, 0)),
    )
    out_shape = (
        jax.ShapeDtypeStruct((n, dx), jnp.float32),
        jax.ShapeDtypeStruct((n, nc), jnp.float32),
        jax.ShapeDtypeStruct((n, nz), jnp.float32),
    )

    args = [x]
    for w, b in zip(weights, b2d):
        args.append(w)
        args.append(b)
    args.append(cluster)

    return pl.pallas_call(
        _fused_body,
        grid=(n // _BLK,),
        in_specs=in_specs,
        out_specs=out_specs,
        out_shape=out_shape,
        compiler_params=pltpu.CompilerParams(
            dimension_semantics=("parallel",),
            vmem_limit_bytes=110 * 1024 * 1024,
        ),
    )(*args)
